# baseline (speedup 1.0000x reference)
import functools
import math

import jax
import jax.numpy as jnp
from jax.experimental import pallas as pl
from jax.experimental.pallas import tpu as pltpu


_ROWS = 4000
_NBUF = 8
_LAG = 4


def _copy_kernel(bank_hbm, q_hbm, phi_ref, scratch, in_sems, out_sems,
                 *, rows, nchunk, nbuf, lag, b, m, dim):
    col = jax.lax.broadcasted_iota(jnp.int32, (1, m), 1).astype(jnp.float32)
    phi_ref[...] = (2.0 * math.pi / m) * col

    prow = rows // 2

    def in_copy(c):
        slot = c % nbuf
        return pltpu.make_async_copy(
            bank_hbm.at[pl.ds(c * prow, prow), :],
            scratch.at[slot],
            in_sems.at[slot],
        )

    def out_copy(c, j):
        slot = c % nbuf
        return pltpu.make_async_copy(
            scratch.at[slot],
            q_hbm.at[j, pl.ds(c * prow, prow), :],
            out_sems.at[slot, j],
        )

    for c in range(min(lag, nchunk)):
        in_copy(c).start()

    unwaited = {}
    for c in range(nchunk):
        in_copy(c).wait()
        for j in range(b):
            out_copy(c, j).start()
        unwaited[c] = True
        r = c + lag
        if r < nchunk:
            prev = r - nbuf
            if prev >= 0 and prev in unwaited:
                for j in range(b):
                    out_copy(prev, j).wait()
                del unwaited[prev]
            in_copy(r).start()
    for c in sorted(unwaited):
        for j in range(b):
            out_copy(c, j).wait()


def kernel(key_embed, bank):
    b = key_embed.shape[0]
    m, dim = bank.shape
    rows = _ROWS
    nchunk = m // rows
    bank_packed = bank.reshape(m // 2, dim * 2)
    q2, phi2d = pl.pallas_call(
        functools.partial(_copy_kernel, rows=rows, nchunk=nchunk,
                          nbuf=_NBUF, lag=_LAG, b=b, m=m, dim=dim),
        in_specs=[pl.BlockSpec(memory_space=pl.ANY)],
        out_specs=[
            pl.BlockSpec(memory_space=pl.ANY),
            pl.BlockSpec(memory_space=pltpu.VMEM),
        ],
        out_shape=[
            jax.ShapeDtypeStruct((b, m // 2, dim * 2), jnp.float32),
            jax.ShapeDtypeStruct((1, m), jnp.float32),
        ],
        scratch_shapes=[
            pltpu.VMEM((_NBUF, _ROWS // 2, 128), jnp.float32),
            pltpu.SemaphoreType.DMA((_NBUF,)),
            pltpu.SemaphoreType.DMA((_NBUF, 4)),
        ],
    )(bank_packed)
    q_valid = jnp.ones((b, m), dtype=bool)
    return (q2.reshape(b, m, dim), q_valid, phi2d.reshape(m))


# speedup vs baseline: 1.2630x; 1.2630x over previous
import functools
import math

import jax
import jax.numpy as jnp
from jax import lax
from jax.experimental import pallas as pl
from jax.experimental.pallas import tpu as pltpu
from jax.experimental.pallas import tpu_sc as plsc


_NWORKERS = 32
_CHUNK = 200


def _sc_replicate_body(bank_hbm, q_hbm, buf, sem, *, b, m, dim):
    c = lax.axis_index("c")
    s = lax.axis_index("s")
    wid = s * 2 + c
    nchunk = m // _CHUNK
    rounds = (nchunk + _NWORKERS - 1) // _NWORKERS
    for k in range(rounds):
        idx = wid + _NWORKERS * k

        @pl.when(idx < nchunk)
        def _():
            r0 = idx * _CHUNK
            pltpu.sync_copy(bank_hbm.at[pl.ds(r0, _CHUNK), :], buf)
            copies = [
                pltpu.make_async_copy(
                    buf, q_hbm.at[j, pl.ds(r0, _CHUNK), :], sem)
                for j in range(b)
            ]
            for cp in copies:
                cp.start()
            for cp in copies:
                cp.wait()


def _phi_kernel(phi_ref, *, m):
    col = jax.lax.broadcasted_iota(jnp.int32, (1, m), 1).astype(jnp.float32)
    phi_ref[...] = (2.0 * math.pi / m) * col


def kernel(key_embed, bank):
    b = key_embed.shape[0]
    m, dim = bank.shape

    mesh = plsc.VectorSubcoreMesh(core_axis_name="c", subcore_axis_name="s")
    q = pl.kernel(
        functools.partial(_sc_replicate_body, b=b, m=m, dim=dim),
        out_type=jax.ShapeDtypeStruct((b, m, dim), jnp.float32),
        mesh=mesh,
        scratch_types=[
            pltpu.VMEM((_CHUNK, dim), jnp.float32),
            pltpu.SemaphoreType.DMA,
        ],
    )(bank)

    phi2d = pl.pallas_call(
        functools.partial(_phi_kernel, m=m),
        out_specs=pl.BlockSpec(memory_space=pltpu.VMEM),
        out_shape=jax.ShapeDtypeStruct((1, m), jnp.float32),
    )()
    q_valid = jnp.ones((b, m), dtype=bool)
    return (q, q_valid, phi2d.reshape(m))


# speedup vs baseline: 1.2997x; 1.0291x over previous
import functools
import math

import jax
import jax.numpy as jnp
from jax import lax
from jax.experimental import pallas as pl
from jax.experimental.pallas import tpu as pltpu
from jax.experimental.pallas import tpu_sc as plsc


_NWORKERS = 32
_CHUNK = 400


def _sc_replicate_body(bank_hbm, q_hbm, bufs, in_sems, out_sems,
                       *, b, m, dim):
    c = lax.axis_index("c")
    s = lax.axis_index("s")
    wid = s * 2 + c
    nchunk = m // _CHUNK
    rounds = (nchunk + _NWORKERS - 1) // _NWORKERS

    def valid(k):
        return (wid + _NWORKERS * k) < nchunk

    def in_copy(k):
        r0 = (wid + _NWORKERS * k) * _CHUNK
        return pltpu.make_async_copy(
            bank_hbm.at[pl.ds(r0, _CHUNK), :], bufs.at[k % 2],
            in_sems.at[k % 2])

    def out_copies(k):
        r0 = (wid + _NWORKERS * k) * _CHUNK
        return [
            pltpu.make_async_copy(
                bufs.at[k % 2], q_hbm.at[j, pl.ds(r0, _CHUNK), :],
                out_sems.at[k % 2])
            for j in range(b)
        ]

    @pl.when(valid(0))
    def _():
        in_copy(0).start()

    for k in range(rounds):
        @pl.when(valid(k))
        def _():
            in_copy(k).wait()
            for cp in out_copies(k):
                cp.start()
        if k >= 1:
            @pl.when(valid(k - 1))
            def _():
                for cp in out_copies(k - 1):
                    cp.wait()
        if k + 1 < rounds:
            @pl.when(valid(k + 1))
            def _():
                in_copy(k + 1).start()

    @pl.when(valid(rounds - 1))
    def _():
        for cp in out_copies(rounds - 1):
            cp.wait()


def _phi_kernel(phi_ref, *, m):
    col = jax.lax.broadcasted_iota(jnp.int32, (1, m), 1).astype(jnp.float32)
    phi_ref[...] = (2.0 * math.pi / m) * col


def kernel(key_embed, bank):
    b = key_embed.shape[0]
    m, dim = bank.shape

    mesh = plsc.VectorSubcoreMesh(core_axis_name="c", subcore_axis_name="s")
    q = pl.kernel(
        functools.partial(_sc_replicate_body, b=b, m=m, dim=dim),
        out_type=jax.ShapeDtypeStruct((b, m, dim), jnp.float32),
        mesh=mesh,
        scratch_types=[
            pltpu.VMEM((2, _CHUNK, dim), jnp.float32),
            pltpu.SemaphoreType.DMA((2,)),
            pltpu.SemaphoreType.DMA((2,)),
        ],
    )(bank)

    phi2d = pl.pallas_call(
        functools.partial(_phi_kernel, m=m),
        out_specs=pl.BlockSpec(memory_space=pltpu.VMEM),
        out_shape=jax.ShapeDtypeStruct((1, m), jnp.float32),
    )()
    q_valid = jnp.ones((b, m), dtype=bool)
    return (q, q_valid, phi2d.reshape(m))


# speedup vs baseline: 1.2998x; 1.0001x over previous
import functools
import math

import jax
import jax.numpy as jnp
from jax import lax
from jax.experimental import pallas as pl
from jax.experimental.pallas import tpu as pltpu
from jax.experimental.pallas import tpu_sc as plsc


_NWORKERS = 32
_CHUNK = 400


def _sc_replicate_body(bank_hbm, q_hbm, bufs, in_sems, out_sems,
                       *, b, m, dim):
    c = lax.axis_index("c")
    s = lax.axis_index("s")
    wid = s * 2 + c
    nchunk = m // _CHUNK
    rounds = (nchunk + _NWORKERS - 1) // _NWORKERS

    def valid(k):
        return (wid + _NWORKERS * k) < nchunk

    def in_copy(k):
        r0 = (wid + _NWORKERS * k) * _CHUNK
        return pltpu.make_async_copy(
            bank_hbm.at[pl.ds(r0, _CHUNK), :], bufs.at[k % 2],
            in_sems.at[k % 2])

    def out_copies(k):
        r0 = (wid + _NWORKERS * k) * _CHUNK
        return [
            pltpu.make_async_copy(
                bufs.at[k % 2], q_hbm.at[j, pl.ds(r0, _CHUNK), :],
                out_sems.at[k % 2])
            for j in range(b)
        ]

    @pl.when(valid(0))
    def _():
        in_copy(0).start()

    for k in range(rounds):
        @pl.when(valid(k))
        def _():
            in_copy(k).wait()
            for cp in out_copies(k):
                cp.start()
        if k >= 1:
            @pl.when(valid(k - 1))
            def _():
                for cp in out_copies(k - 1):
                    cp.wait()
        if k + 1 < rounds:
            @pl.when(valid(k + 1))
            def _():
                in_copy(k + 1).start()

    @pl.when(valid(rounds - 1))
    def _():
        for cp in out_copies(rounds - 1):
            cp.wait()


def _phi_kernel(phi_ref, *, m):
    col = jax.lax.broadcasted_iota(jnp.int32, (1, m), 1).astype(jnp.float32)
    phi_ref[...] = (2.0 * math.pi / m) * col


def kernel(key_embed, bank):
    b = key_embed.shape[0]
    m, dim = bank.shape

    mesh = plsc.VectorSubcoreMesh(core_axis_name="c", subcore_axis_name="s")
    q = pl.kernel(
        functools.partial(_sc_replicate_body, b=b, m=m, dim=dim),
        out_type=jax.ShapeDtypeStruct((b, m, dim), jnp.float32),
        mesh=mesh,
        scratch_types=[
            pltpu.VMEM((2, _CHUNK, dim), jnp.float32),
            pltpu.SemaphoreType.DMA((2,)),
            pltpu.SemaphoreType.DMA((2,)),
        ],
        compiler_params=pltpu.CompilerParams(use_tc_tiling_on_sc=True),
    )(bank)

    phi2d = pl.pallas_call(
        functools.partial(_phi_kernel, m=m),
        out_specs=pl.BlockSpec(memory_space=pltpu.VMEM),
        out_shape=jax.ShapeDtypeStruct((1, m), jnp.float32),
    )()
    q_valid = jnp.ones((b, m), dtype=bool)
    return (q, q_valid, phi2d.reshape(m))


# speedup vs baseline: 8.2473x; 6.3448x over previous
import functools
import math

import jax
import jax.numpy as jnp
from jax.experimental import pallas as pl
from jax.experimental.pallas import tpu as pltpu


_RCHUNK = 8
_NBUF = 4
_LAG = 2


def _rep_kernel(bank_hbm, q_hbm, phi_ref, scratch, in_sems, out_sems,
                *, rows, nchunk, nbuf, lag, b, m):
    col = jax.lax.broadcasted_iota(jnp.int32, (1, m), 1).astype(jnp.float32)
    phi_ref[...] = (2.0 * math.pi / m) * col

    def in_copy(c):
        slot = c % nbuf
        return pltpu.make_async_copy(
            bank_hbm.at[pl.ds(c * rows, rows), :],
            scratch.at[slot],
            in_sems.at[slot],
        )

    def out_copy(c, j):
        slot = c % nbuf
        return pltpu.make_async_copy(
            scratch.at[slot],
            q_hbm.at[j, pl.ds(c * rows, rows), :],
            out_sems.at[slot, j],
        )

    for c in range(min(lag, nchunk)):
        in_copy(c).start()

    unwaited = {}
    for c in range(nchunk):
        in_copy(c).wait()
        for j in range(b):
            out_copy(c, j).start()
        unwaited[c] = True
        r = c + lag
        if r < nchunk:
            prev = r - nbuf
            if prev >= 0 and prev in unwaited:
                for j in range(b):
                    out_copy(prev, j).wait()
                del unwaited[prev]
            in_copy(r).start()
    for c in sorted(unwaited):
        for j in range(b):
            out_copy(c, j).wait()


def kernel(key_embed, bank):
    b = key_embed.shape[0]
    m, dim = bank.shape
    rows = _RCHUNK
    nchunk = dim // rows
    bank_t = bank.T
    qt, phi2d = pl.pallas_call(
        functools.partial(_rep_kernel, rows=rows, nchunk=nchunk,
                          nbuf=_NBUF, lag=_LAG, b=b, m=m),
        in_specs=[pl.BlockSpec(memory_space=pl.ANY)],
        out_specs=[
            pl.BlockSpec(memory_space=pl.ANY),
            pl.BlockSpec(memory_space=pltpu.VMEM),
        ],
        out_shape=[
            jax.ShapeDtypeStruct((b, dim, m), jnp.float32),
            jax.ShapeDtypeStruct((1, m), jnp.float32),
        ],
        scratch_shapes=[
            pltpu.VMEM((_NBUF, _RCHUNK, m), jnp.float32),
            pltpu.SemaphoreType.DMA((_NBUF,)),
            pltpu.SemaphoreType.DMA((_NBUF, 4)),
        ],
    )(bank_t)
    q = qt.transpose(0, 2, 1)
    q_valid = jnp.ones((b, m), dtype=bool)
    return (q, q_valid, phi2d.reshape(m))
